# baseline (device time: 31565 ns/iter reference)
import jax
import jax.numpy as jnp
from jax import lax
from jax.experimental import pallas as pl
from jax.experimental.pallas import tpu as pltpu

N_DEV = 32
B = 2
SQ = 128
HQ = 8
HKV = 2
DH = 64
D = HQ * DH
DL = D + HQ
G = HQ // HKV
SCALE = 0.125
RPS = (B * SQ) // N_DEV


def kernel(x, Wq, Wo, K_ext, V_ext):
    def body(x_ref, wq_ref, wo_ref, k_ref, v_ref, out_ref,
             ol_send, ol_rs, out_slab, out2,
             send_1, recv_1, send_2, recv_2):
        my = lax.axis_index("i")

        barrier = pltpu.get_barrier_semaphore()
        for d in range(N_DEV):
            pl.semaphore_signal(
                barrier, inc=1, device_id=(d,),
                device_id_type=pl.DeviceIdType.MESH)
        pl.semaphore_wait(barrier, N_DEV)

        def make_1(d):
            return pltpu.make_async_remote_copy(
                src_ref=ol_send.at[pl.ds(d * RPS, RPS), :],
                dst_ref=ol_rs.at[pl.ds(my * RPS, RPS), :],
                send_sem=send_1.at[d], recv_sem=recv_1.at[my],
                device_id=(d,), device_id_type=pl.DeviceIdType.MESH)

        def make_2(d):
            return pltpu.make_async_remote_copy(
                src_ref=out_slab,
                dst_ref=out2.at[pl.ds(my * RPS, RPS), :],
                send_sem=send_2.at[d], recv_sem=recv_2.at[my],
                device_id=(d,), device_id_type=pl.DeviceIdType.MESH)

        rdma1 = {}

        wq16 = wq_ref[:, :].astype(jnp.bfloat16)
        for b in range(B):
            qb = jnp.dot(x_ref[b].astype(jnp.bfloat16), wq16,
                         preferred_element_type=jnp.float32)
            for g in range(HKV):
                k = k_ref[b, :, g, :].astype(jnp.bfloat16)
                v = v_ref[b, :, g, :].astype(jnp.bfloat16)
                qs = jnp.concatenate(
                    [qb[:, (g * G + hh) * DH:(g * G + hh + 1) * DH]
                     for hh in range(G)], axis=0).astype(jnp.bfloat16)
                s = lax.dot_general(
                    qs, k, (((1,), (1,)), ((), ())),
                    preferred_element_type=jnp.float32) * SCALE
                p = jnp.exp(s)
                l = jnp.sum(p, axis=1, keepdims=True)
                o4 = jnp.dot(p.astype(jnp.bfloat16), v,
                             preferred_element_type=jnp.float32)
                for hh in range(G):
                    h = g * G + hh
                    rows = pl.ds(b * SQ, SQ)
                    ol_send[rows, h * DH:(h + 1) * DH] = (
                        o4[hh * SQ:(hh + 1) * SQ, :])
                    ol_send[rows, D + h:D + h + 1] = l[hh * SQ:(hh + 1) * SQ, :]
            for d in range(b * (N_DEV // B), (b + 1) * (N_DEV // B)):
                rdma1[d] = make_1(d)
                rdma1[d].start()

        for s in range(N_DEV):
            dsc = pltpu.make_async_remote_copy(
                src_ref=ol_rs.at[pl.ds(s * RPS, RPS), :],
                dst_ref=ol_rs.at[pl.ds(s * RPS, RPS), :],
                send_sem=send_1.at[s], recv_sem=recv_1.at[s],
                device_id=(0,), device_id_type=pl.DeviceIdType.MESH)
            dsc.wait_recv()

        terms = [ol_rs[s * RPS:(s + 1) * RPS, :] for s in range(N_DEV)]
        while len(terms) > 1:
            terms = [terms[i] + terms[i + 1] for i in range(0, len(terms), 2)]
        acc = terms[0]
        l_sum = acc[:, D:]
        l_e = jnp.concatenate(
            [jnp.broadcast_to(l_sum[:, h:h + 1], (RPS, DH))
             for h in range(HQ)], axis=1)
        out_slab[:, :] = jnp.dot(
            (acc[:, :D] / l_e).astype(jnp.bfloat16),
            wo_ref[:, :].astype(jnp.bfloat16),
            preferred_element_type=jnp.float32)

        rdma2 = {}
        for d in range(N_DEV):
            rdma2[d] = make_2(d)
            rdma2[d].start()
        for s in range(N_DEV):
            dsc = pltpu.make_async_remote_copy(
                src_ref=out2.at[pl.ds(s * RPS, RPS), :],
                dst_ref=out2.at[pl.ds(s * RPS, RPS), :],
                send_sem=send_2.at[s], recv_sem=recv_2.at[s],
                device_id=(0,), device_id_type=pl.DeviceIdType.MESH)
            dsc.wait_recv()

        for b in range(B):
            out_ref[b] = out2[b * SQ:(b + 1) * SQ, :]

        for d in range(N_DEV):
            rdma1[d].wait_send()
            rdma2[d].wait_send()

    return pl.pallas_call(
        body,
        out_shape=jax.ShapeDtypeStruct((B, SQ, D), jnp.float32),
        in_specs=[pl.BlockSpec(memory_space=pltpu.VMEM)] * 5,
        out_specs=pl.BlockSpec(memory_space=pltpu.VMEM),
        scratch_shapes=[
            pltpu.VMEM((B * SQ, DL), jnp.float32),
            pltpu.VMEM((B * SQ, DL), jnp.float32),
            pltpu.VMEM((RPS, D), jnp.float32),
            pltpu.VMEM((B * SQ, D), jnp.float32),
            pltpu.SemaphoreType.DMA((N_DEV,)),
            pltpu.SemaphoreType.DMA((N_DEV,)),
            pltpu.SemaphoreType.DMA((N_DEV,)),
            pltpu.SemaphoreType.DMA((N_DEV,)),
        ],
        compiler_params=pltpu.CompilerParams(collective_id=0),
    )(x, Wq, Wo, K_ext, V_ext)
